# Initial kernel scaffold; baseline (speedup 1.0000x reference)
import jax
import jax.numpy as jnp
from jax.experimental import pallas as pl


def kernel(x, A):
    raise NotImplementedError("write your pallas kernel here")


# speedup vs baseline: 82.3107x; 82.3107x over previous
import functools

import jax
import jax.numpy as jnp
from jax import lax
from jax.experimental import pallas as pl

BST_DEPTH = 10
NB_SPLIT = 2**BST_DEPTH - 1
NB_NODES = 2**(BST_DEPTH + 1) - 1


def _interleave(left, right):
    tb, w = left.shape
    h = min(w, 64)
    cw = 2 * h
    chunks = []
    i = lax.broadcasted_iota(jnp.int32, (tb, cw), 1)
    src = (i >> 1) + (i % 2) * h
    for k in range(w // h):
        t = jnp.concatenate([left[:, h * k:h * (k + 1)],
                             right[:, h * k:h * (k + 1)]], axis=1)
        chunks.append(jnp.take_along_axis(t, src, axis=1))
    return chunks[0] if len(chunks) == 1 else jnp.concatenate(chunks, axis=1)


def _lpsparse_kernel(x_ref, a_ref, o_ref):
    xa = lax.dot_general(
        x_ref[...], a_ref[...],
        dimension_numbers=(((1,), (1,)), ((), ())),
        preferred_element_type=jnp.float32,
    )
    tb = xa.shape[0]
    p = jnp.ones((tb, 1), jnp.float32)
    o_ref[:, 0:1] = p
    off = 1
    for d in range(BST_DEPTH):
        w = 2**d
        s = xa[:, w - 1:2 * w - 1]
        p = _interleave(jnp.minimum(p, s), jnp.minimum(p, -s))
        o_ref[:, off:off + 2 * w] = jnp.clip(p, 0.0, 1.0)
        off += 2 * w


@functools.partial(jax.jit, static_argnames=("tb",))
def _run(x, a_pad, tb=256):
    batch, dim = x.shape
    grid = (batch // tb,)
    return pl.pallas_call(
        _lpsparse_kernel,
        grid=grid,
        in_specs=[
            pl.BlockSpec((tb, dim), lambda i: (i, 0)),
            pl.BlockSpec((a_pad.shape[0], dim), lambda i: (0, 0)),
        ],
        out_specs=pl.BlockSpec((tb, NB_NODES), lambda i: (i, 0)),
        out_shape=jax.ShapeDtypeStruct((batch, NB_NODES), jnp.float32),
    )(x, a_pad)


def kernel(x, A):
    a_pad = jnp.pad(A, ((0, 1), (0, 0)))
    return _run(x, a_pad)


# speedup vs baseline: 101.0727x; 1.2279x over previous
import functools

import jax
import jax.numpy as jnp
from jax import lax
from jax.experimental import pallas as pl

BST_DEPTH = 10
NB_NODES = 2**(BST_DEPTH + 1) - 1
LEVEL_OFF = [0, 1, 3, 7, 15, 31, 63, 128, 256, 512]


def _lpsparse_kernel(x_ref, a_ref, o_ref):
    xa = lax.dot_general(
        x_ref[...], a_ref[...],
        dimension_numbers=(((1,), (1,)), ((), ())),
        preferred_element_type=jnp.float32,
    )
    tb = xa.shape[0]
    i128 = lax.broadcasted_iota(jnp.int32, (tb, 128), 1)
    g128 = i128 // 2
    sgn128 = jnp.where(i128 % 2 == 0, 1.0, -1.0)

    p = jnp.ones((tb, 1), jnp.float32)
    o_ref[:, 0:1] = p
    off = 1
    for d in range(BST_DEPTH):
        w = 2**d
        n = 2 * w
        o = LEVEL_OFF[d]
        if n <= 128:
            i = lax.broadcasted_iota(jnp.int32, (tb, n), 1)
            ss = jnp.take_along_axis(xa[:, :128], o + i // 2, axis=1)
            sp = jnp.take_along_axis(p, i // 2, axis=1)
            sgn = jnp.where(i % 2 == 0, 1.0, -1.0)
            p = jnp.minimum(sp, ss * sgn)
        else:
            chunks = []
            for j in range(n // 128):
                base = 128 * (j // 2)
                idx = g128 + 64 * (j % 2)
                ssj = jnp.take_along_axis(xa[:, o + base:o + base + 128], idx,
                                          axis=1)
                spj = jnp.take_along_axis(p[:, base:base + 128], idx, axis=1)
                chunks.append(jnp.minimum(spj, ssj * sgn128))
            p = jnp.concatenate(chunks, axis=1)
        o_ref[:, off:off + n] = jnp.clip(p, 0.0, 1.0)
        off += n


@functools.partial(jax.jit, static_argnames=("tb",))
def _run(x, a_pad, tb=256):
    batch, dim = x.shape
    grid = (batch // tb,)
    return pl.pallas_call(
        _lpsparse_kernel,
        grid=grid,
        in_specs=[
            pl.BlockSpec((tb, dim), lambda i: (i, 0)),
            pl.BlockSpec((a_pad.shape[0], dim), lambda i: (0, 0)),
        ],
        out_specs=pl.BlockSpec((tb, NB_NODES), lambda i: (i, 0)),
        out_shape=jax.ShapeDtypeStruct((batch, NB_NODES), jnp.float32),
    )(x, a_pad)


def kernel(x, A):
    a_pad = jnp.concatenate(
        [A[:127], jnp.zeros((1, A.shape[1]), A.dtype), A[127:]], axis=0)
    return _run(x, a_pad)


# speedup vs baseline: 110.5397x; 1.0937x over previous
import functools

import jax
import jax.numpy as jnp
from jax import lax
from jax.experimental import pallas as pl

BST_DEPTH = 10
NB_NODES = 2**(BST_DEPTH + 1) - 1
LEVEL_OFF = [0, 1, 3, 7, 15, 31, 63, 128, 256, 512]


def _tree_half(xa, o_ref, r0, rows):
    i128 = lax.broadcasted_iota(jnp.int32, (rows, 128), 1)
    g128 = i128 // 2
    sgn128 = jnp.where(i128 % 2 == 0, 1.0, -1.0)

    p = jnp.ones((rows, 1), jnp.float32)
    o_ref[r0:r0 + rows, 0:1] = p
    off = 1
    for d in range(BST_DEPTH):
        w = 2**d
        n = 2 * w
        o = LEVEL_OFF[d]
        if n <= 128:
            i = lax.broadcasted_iota(jnp.int32, (rows, n), 1)
            ss = jnp.take_along_axis(xa[:, :128], o + i // 2, axis=1)
            sp = jnp.take_along_axis(p, i // 2, axis=1)
            sgn = jnp.where(i % 2 == 0, 1.0, -1.0)
            p = jnp.minimum(sp, ss * sgn)
        else:
            chunks = []
            for j in range(n // 128):
                base = 128 * (j // 2)
                idx = g128 + 64 * (j % 2)
                ssj = jnp.take_along_axis(xa[:, o + base:o + base + 128], idx,
                                          axis=1)
                spj = jnp.take_along_axis(p[:, base:base + 128], idx, axis=1)
                chunks.append(jnp.minimum(spj, ssj * sgn128))
            p = jnp.concatenate(chunks, axis=1)
        o_ref[r0:r0 + rows, off:off + n] = jnp.clip(p, 0.0, 1.0)
        off += n


def _lpsparse_kernel(x_ref, a_ref, o_ref, *, halves):
    tb = x_ref.shape[0]
    hh = tb // halves
    for h in range(halves):
        xa = lax.dot_general(
            x_ref[h * hh:(h + 1) * hh, :], a_ref[...],
            dimension_numbers=(((1,), (1,)), ((), ())),
            preferred_element_type=jnp.float32,
        )
        _tree_half(xa, o_ref, h * hh, hh)


@functools.partial(jax.jit, static_argnames=("tb", "halves"))
def _run(x, a_pad, tb=1024, halves=1):
    batch, dim = x.shape
    grid = (batch // tb,)
    return pl.pallas_call(
        functools.partial(_lpsparse_kernel, halves=halves),
        grid=grid,
        in_specs=[
            pl.BlockSpec((tb, dim), lambda i: (i, 0)),
            pl.BlockSpec((a_pad.shape[0], dim), lambda i: (0, 0)),
        ],
        out_specs=pl.BlockSpec((tb, NB_NODES), lambda i: (i, 0)),
        out_shape=jax.ShapeDtypeStruct((batch, NB_NODES), jnp.float32),
    )(x, a_pad)


def kernel(x, A):
    a_pad = jnp.concatenate(
        [A[:127], jnp.zeros((1, A.shape[1]), A.dtype), A[127:]], axis=0)
    return _run(x, a_pad)


# speedup vs baseline: 129.1962x; 1.1688x over previous
import functools

import jax
import jax.numpy as jnp
from jax import lax
from jax.experimental import pallas as pl
from jax.experimental.pallas import tpu as pltpu

BST_DEPTH = 10
NB_NODES = 2**(BST_DEPTH + 1) - 1
LEVEL_OFF = [0, 1, 3, 7, 15, 31, 63, 128, 256, 512]


def _tree_half(xa, o_ref, r0, rows):
    i128 = lax.broadcasted_iota(jnp.int32, (rows, 128), 1)
    g128 = i128 // 2
    sgn128 = jnp.where(i128 % 2 == 0, 1.0, -1.0)

    p = jnp.ones((rows, 1), jnp.float32)
    o_ref[r0:r0 + rows, 0:1] = p
    off = 1
    for d in range(BST_DEPTH):
        w = 2**d
        n = 2 * w
        o = LEVEL_OFF[d]
        if n <= 128:
            i = lax.broadcasted_iota(jnp.int32, (rows, n), 1)
            ss = jnp.take_along_axis(xa[:, :128], o + i // 2, axis=1)
            sp = jnp.take_along_axis(p, i // 2, axis=1)
            sgn = jnp.where(i % 2 == 0, 1.0, -1.0)
            p = jnp.minimum(sp, ss * sgn)
        else:
            chunks = []
            for j in range(n // 128):
                base = 128 * (j // 2)
                idx = g128 + 64 * (j % 2)
                ssj = jnp.take_along_axis(xa[:, o + base:o + base + 128], idx,
                                          axis=1)
                spj = jnp.take_along_axis(p[:, base:base + 128], idx, axis=1)
                chunks.append(jnp.minimum(spj, ssj * sgn128))
            p = jnp.concatenate(chunks, axis=1)
        o_ref[r0:r0 + rows, off:off + n] = jnp.clip(p, 0.0, 1.0)
        off += n


def _lpsparse_kernel(x_ref, a_ref, o_ref, xa_scr, *, ntiles):
    i = pl.program_id(0)
    tb = x_ref.shape[0]

    xa_prev = xa_scr[pl.ds(((i - 1) % 2) * tb, tb), :]
    xa = lax.dot_general(
        x_ref[...], a_ref[...],
        dimension_numbers=(((1,), (1,)), ((), ())),
        preferred_element_type=jnp.float32,
    )
    xa_scr[pl.ds((i % 2) * tb, tb), :] = xa
    _tree_half(xa_prev, o_ref, 0, tb)


@functools.partial(jax.jit, static_argnames=("tb",))
def _run(x, a_pad, tb=256):
    batch, dim = x.shape
    ntiles = batch // tb
    return pl.pallas_call(
        functools.partial(_lpsparse_kernel, ntiles=ntiles),
        grid=(ntiles + 1,),
        in_specs=[
            pl.BlockSpec((tb, dim), lambda i: (jnp.minimum(i, ntiles - 1), 0)),
            pl.BlockSpec((a_pad.shape[0], dim), lambda i: (0, 0)),
        ],
        out_specs=pl.BlockSpec((tb, NB_NODES),
                               lambda i: (jnp.maximum(i - 1, 0), 0)),
        out_shape=jax.ShapeDtypeStruct((batch, NB_NODES), jnp.float32),
        scratch_shapes=[pltpu.VMEM((2 * tb, 1024), jnp.float32)],
    )(x, a_pad)


def kernel(x, A):
    a_pad = jnp.concatenate(
        [A[:127], jnp.zeros((1, A.shape[1]), A.dtype), A[127:]], axis=0)
    return _run(x, a_pad)


# speedup vs baseline: 138.7834x; 1.0742x over previous
import functools

import jax
import jax.numpy as jnp
from jax import lax
from jax.experimental import pallas as pl
from jax.experimental.pallas import tpu as pltpu

BST_DEPTH = 10
NB_NODES = 2**(BST_DEPTH + 1) - 1
LEVEL_OFF = [0, 1, 3, 7, 15, 31, 63, 128, 256, 512]


def _tree_half(xa, o_ref, r0, rows):
    i128 = lax.broadcasted_iota(jnp.int32, (rows, 128), 1)
    g128 = i128 // 2
    sgn128 = jnp.where(i128 % 2 == 0, 1.0, -1.0)

    p = jnp.ones((rows, 1), jnp.float32)
    o_ref[r0:r0 + rows, 0:1] = p
    off = 1
    for d in range(BST_DEPTH):
        w = 2**d
        n = 2 * w
        o = LEVEL_OFF[d]
        if n <= 128:
            i = lax.broadcasted_iota(jnp.int32, (rows, n), 1)
            ss = jnp.take_along_axis(xa[:, :128], o + i // 2, axis=1)
            sp = jnp.take_along_axis(p, i // 2, axis=1)
            sgn = jnp.where(i % 2 == 0, 1.0, -1.0)
            p = jnp.minimum(sp, ss * sgn)
        else:
            chunks = []
            for j in range(n // 128):
                base = 128 * (j // 2)
                idx = g128 + 64 * (j % 2)
                ssj = jnp.take_along_axis(xa[:, o + base:o + base + 128], idx,
                                          axis=1)
                spj = jnp.take_along_axis(p[:, base:base + 128], idx, axis=1)
                chunks.append(jnp.minimum(spj, ssj * sgn128))
            p = jnp.concatenate(chunks, axis=1)
        o_ref[r0:r0 + rows, off:off + n] = jnp.clip(p, 0.0, 1.0)
        off += n


def _lpsparse_kernel(x_ref, a_ref, o_ref, xa_scr, *, ntiles):
    i = pl.program_id(0)
    tb = x_ref.shape[0]

    xa_prev = xa_scr[pl.ds(((i - 1) % 2) * tb, tb), :]
    xa = lax.dot_general(
        x_ref[...], a_ref[...],
        dimension_numbers=(((1,), (1,)), ((), ())),
        preferred_element_type=jnp.float32,
    )
    xa_scr[pl.ds((i % 2) * tb, tb), :] = xa
    _tree_half(xa_prev, o_ref, 0, tb)


@functools.partial(jax.jit, static_argnames=("tb",))
def _run(x, a_pad, tb=512):
    batch, dim = x.shape
    ntiles = batch // tb
    return pl.pallas_call(
        functools.partial(_lpsparse_kernel, ntiles=ntiles),
        grid=(ntiles + 1,),
        in_specs=[
            pl.BlockSpec((tb, dim), lambda i: (jnp.minimum(i, ntiles - 1), 0)),
            pl.BlockSpec((a_pad.shape[0], dim), lambda i: (0, 0)),
        ],
        out_specs=pl.BlockSpec((tb, NB_NODES),
                               lambda i: (jnp.maximum(i - 1, 0), 0)),
        out_shape=jax.ShapeDtypeStruct((batch, NB_NODES), jnp.float32),
        scratch_shapes=[pltpu.VMEM((2 * tb, 1024), jnp.float32)],
    )(x, a_pad)


def kernel(x, A):
    a_pad = jnp.concatenate(
        [A[:127], jnp.zeros((1, A.shape[1]), A.dtype), A[127:]], axis=0)
    return _run(x, a_pad)


# speedup vs baseline: 162.7229x; 1.1725x over previous
import functools

import jax
import jax.numpy as jnp
from jax import lax
from jax.experimental import pallas as pl
from jax.experimental.pallas import tpu as pltpu

BST_DEPTH = 10
NB_NODES = 2**(BST_DEPTH + 1) - 1
LEVEL_OFF = [0, 1, 3, 7, 15, 31, 63, 128, 256, 512]


def _tree_half(xa, o_ref, r0, rows):
    i128 = lax.broadcasted_iota(jnp.int32, (rows, 128), 1)
    g128 = i128 // 2
    sgn128 = jnp.where(i128 % 2 == 0, 1.0, -1.0)

    c0 = xa[:, :128]
    par = jnp.maximum((i128 - 1) // 2, 0)
    sgn_odd = jnp.where(i128 % 2 == 1, 1.0, -1.0)
    e0 = jnp.where(i128 == 0, 1.0,
                   sgn_odd * jnp.take_along_axis(c0, par, axis=1))
    p2 = jnp.maximum((par - 1) // 2, 0)
    p4 = jnp.maximum((p2 - 1) // 2, 0)
    p4 = jnp.maximum((p4 - 1) // 2, 0)
    q0 = jnp.minimum(e0, jnp.take_along_axis(e0, par, axis=1))
    q0 = jnp.minimum(q0, jnp.take_along_axis(q0, p2, axis=1))
    q0 = jnp.minimum(q0, jnp.take_along_axis(q0, p4, axis=1))
    n1 = i128 + 128
    par1 = (n1 - 1) // 2
    sgn1 = jnp.where(n1 % 2 == 1, 1.0, -1.0)
    e1 = sgn1 * jnp.take_along_axis(c0, par1, axis=1)
    q1 = jnp.minimum(e1, jnp.take_along_axis(q0, par1, axis=1))
    o_ref[r0:r0 + rows, 0:128] = jnp.clip(q0, 0.0, 1.0)
    o_ref[r0:r0 + rows, 128:255] = jnp.clip(q1[:, :127], 0.0, 1.0)
    q0_127 = jnp.take_along_axis(q0, jnp.full((rows, 128), 127, jnp.int32),
                                 axis=1)
    p = jnp.where(i128 == 0, q0_127, pltpu.roll(q1, 1, 1))

    off = 255
    for d in range(7, BST_DEPTH):
        w = 2**d
        n = 2 * w
        o = LEVEL_OFF[d]
        chunks = []
        for j in range(n // 128):
            base = 128 * (j // 2)
            idx = g128 + 64 * (j % 2)
            ssj = jnp.take_along_axis(xa[:, o + base:o + base + 128], idx,
                                      axis=1)
            spj = jnp.take_along_axis(p[:, base:base + 128], idx, axis=1)
            chunks.append(jnp.minimum(spj, ssj * sgn128))
        p = jnp.concatenate(chunks, axis=1)
        o_ref[r0:r0 + rows, off:off + n] = jnp.clip(p, 0.0, 1.0)
        off += n


def _lpsparse_kernel(x_ref, a_ref, o_ref, xa_scr, *, ntiles):
    i = pl.program_id(0)
    tb = x_ref.shape[0]

    xa_prev = xa_scr[pl.ds(((i - 1) % 2) * tb, tb), :]
    xa = lax.dot_general(
        x_ref[...], a_ref[...],
        dimension_numbers=(((1,), (1,)), ((), ())),
        preferred_element_type=jnp.float32,
    )
    xa_scr[pl.ds((i % 2) * tb, tb), :] = xa
    _tree_half(xa_prev, o_ref, 0, tb)


@functools.partial(jax.jit, static_argnames=("tb",))
def _run(x, a_pad, tb=512):
    batch, dim = x.shape
    ntiles = batch // tb
    return pl.pallas_call(
        functools.partial(_lpsparse_kernel, ntiles=ntiles),
        grid=(ntiles + 1,),
        in_specs=[
            pl.BlockSpec((tb, dim), lambda i: (jnp.minimum(i, ntiles - 1), 0)),
            pl.BlockSpec((a_pad.shape[0], dim), lambda i: (0, 0)),
        ],
        out_specs=pl.BlockSpec((tb, NB_NODES),
                               lambda i: (jnp.maximum(i - 1, 0), 0)),
        out_shape=jax.ShapeDtypeStruct((batch, NB_NODES), jnp.float32),
        scratch_shapes=[pltpu.VMEM((2 * tb, 1024), jnp.float32)],
    )(x, a_pad)


def kernel(x, A):
    a_pad = jnp.concatenate(
        [A[:127], jnp.zeros((1, A.shape[1]), A.dtype), A[127:]], axis=0)
    return _run(x, a_pad)


# speedup vs baseline: 162.9222x; 1.0012x over previous
import functools

import jax
import jax.numpy as jnp
from jax import lax
from jax.experimental import pallas as pl
from jax.experimental.pallas import tpu as pltpu

BST_DEPTH = 10
NB_NODES = 2**(BST_DEPTH + 1) - 1
LEVEL_OFF = [0, 1, 3, 7, 15, 31, 63, 128, 256, 512]


def _tree_half(xa, o_ref, r0, rows):
    i128 = lax.broadcasted_iota(jnp.int32, (rows, 128), 1)
    g128 = i128 // 2
    sgn128 = jnp.where(i128 % 2 == 0, 1.0, -1.0)

    c0 = xa[:, :128]
    par = jnp.maximum((i128 - 1) // 2, 0)
    sgn_odd = jnp.where(i128 % 2 == 1, 1.0, -1.0)
    e0 = jnp.where(i128 == 0, 1.0,
                   sgn_odd * jnp.take_along_axis(c0, par, axis=1))
    p2 = jnp.maximum((par - 1) // 2, 0)
    p4 = jnp.maximum((p2 - 1) // 2, 0)
    p4 = jnp.maximum((p4 - 1) // 2, 0)
    q0 = jnp.minimum(e0, jnp.take_along_axis(e0, par, axis=1))
    q0 = jnp.minimum(q0, jnp.take_along_axis(q0, p2, axis=1))
    q0 = jnp.minimum(q0, jnp.take_along_axis(q0, p4, axis=1))
    n1 = i128 + 128
    par1 = (n1 - 1) // 2
    sgn1 = jnp.where(n1 % 2 == 1, 1.0, -1.0)
    e1 = sgn1 * jnp.take_along_axis(c0, par1, axis=1)
    q1 = jnp.minimum(e1, jnp.take_along_axis(q0, par1, axis=1))
    o_ref[r0:r0 + rows, 0:128] = jnp.clip(q0, 0.0, 1.0)
    o_ref[r0:r0 + rows, 128:255] = jnp.clip(q1[:, :127], 0.0, 1.0)
    q0_127 = jnp.take_along_axis(q0, jnp.full((rows, 128), 127, jnp.int32),
                                 axis=1)
    p = [jnp.where(i128 == 0, q0_127, pltpu.roll(q1, 1, 1))]

    off = 255
    for d in range(7, BST_DEPTH):
        n = 2**(d + 1)
        o = LEVEL_OFF[d]
        chunks = []
        for j in range(n // 128):
            idx = g128 + 64 * (j % 2)
            src = xa[:, o + 128 * (j // 2):o + 128 * (j // 2) + 128]
            ssj = jnp.take_along_axis(src, idx, axis=1)
            spj = jnp.take_along_axis(p[j // 2], idx, axis=1)
            chunks.append(jnp.minimum(spj, ssj * sgn128))
        o_ref[r0:r0 + rows, off:off + n] = (
            jnp.clip(jnp.concatenate(chunks, axis=1), 0.0, 1.0))
        p = chunks
        off += n


def _lpsparse_kernel(x_ref, a_ref, o_ref, xa_scr, *, ntiles):
    i = pl.program_id(0)
    tb = x_ref.shape[0]

    xa_prev = xa_scr[pl.ds(((i - 1) % 2) * tb, tb), :]
    xa = lax.dot_general(
        x_ref[...], a_ref[...],
        dimension_numbers=(((1,), (1,)), ((), ())),
        preferred_element_type=jnp.float32,
    )
    xa_scr[pl.ds((i % 2) * tb, tb), :] = xa
    _tree_half(xa_prev, o_ref, 0, tb)


@functools.partial(jax.jit, static_argnames=("tb",))
def _run(x, a_pad, tb=512):
    batch, dim = x.shape
    ntiles = batch // tb
    return pl.pallas_call(
        functools.partial(_lpsparse_kernel, ntiles=ntiles),
        grid=(ntiles + 1,),
        in_specs=[
            pl.BlockSpec((tb, dim), lambda i: (jnp.minimum(i, ntiles - 1), 0)),
            pl.BlockSpec((a_pad.shape[0], dim), lambda i: (0, 0)),
        ],
        out_specs=pl.BlockSpec((tb, NB_NODES),
                               lambda i: (jnp.maximum(i - 1, 0), 0)),
        out_shape=jax.ShapeDtypeStruct((batch, NB_NODES), jnp.float32),
        scratch_shapes=[pltpu.VMEM((2 * tb, 1024), jnp.float32)],
    )(x, a_pad)


def kernel(x, A):
    a_pad = jnp.concatenate(
        [A[:127], jnp.zeros((1, A.shape[1]), A.dtype), A[127:]], axis=0)
    return _run(x, a_pad)
